# baseline (speedup 1.0000x reference)
import functools
import numpy as np
import jax
import jax.numpy as jnp
from jax import lax
from jax.experimental import pallas as pl
from jax.experimental.pallas import tpu as pltpu
from jax.experimental.pallas import tpu_sc as plsc

NC, NS, L = 2, 16, 16
NW = NC * NS


def _pe_table(seq_len, d_model):
    position = np.arange(seq_len, dtype=np.float32)[:, None]
    div_term = np.exp(
        np.arange(0, d_model, 2, dtype=np.float32) * (-np.log(10000.0) / d_model)
    )
    pe = np.zeros((seq_len, d_model), dtype=np.float32)
    pe[:, 0::2] = np.sin(position * div_term)
    pe[:, 1::2] = np.cos(position * div_term)
    return pe


def _rowid_body(idx_ref, out_ref):
    ts = out_ref.shape[0]
    i = pl.program_id(0)
    tvec = i * ts + lax.broadcasted_iota(jnp.int32, (ts, 1), 0)
    g = tvec * 5
    out_ref[:, 0:1] = g
    idxb = idx_ref[...]
    out_ref[:, 1:9] = jnp.where(idxb < 4, g + 1 + idxb, 40960)


def kernel(temporal_block, mod_emb_weight, mask_token, temporal_revert_idx,
           temporal_masked_idx):
    b, s, m1, d = temporal_block.shape
    r = temporal_revert_idx.shape[-1]
    n = b * s
    nrow = n * (r + 1)

    table = jnp.concatenate(
        [temporal_block.reshape(n * m1, d), mask_token], axis=0
    )
    idx = temporal_revert_idx.reshape(n, r).astype(jnp.int32)
    pe = jnp.asarray(_pe_table(s, d))
    mod9 = mod_emb_weight[: r + 1]

    ts = 256
    rowids = pl.pallas_call(
        _rowid_body,
        grid=(n // ts,),
        in_specs=[pl.BlockSpec((ts, r), lambda i: (i, 0))],
        out_specs=pl.BlockSpec((ts, r + 1), lambda i: (i, 0)),
        out_shape=jax.ShapeDtypeStruct((n, r + 1), jnp.int32),
    )(idx).reshape(nrow)

    rpw = nrow // NW
    tpc = 8
    ch = tpc * (r + 1)
    nch = rpw // ch
    tpw = n // NW
    mesh = plsc.VectorSubcoreMesh(core_axis_name="c", subcore_axis_name="s")

    @functools.partial(
        pl.kernel,
        mesh=mesh,
        out_type=jax.ShapeDtypeStruct((nrow, d), jnp.float32),
        scratch_types=[
            pltpu.VMEM((ch,), jnp.int32),
            pltpu.VMEM((ch, d), jnp.float32),
            pltpu.VMEM((r + 1, d), jnp.float32),
            pltpu.VMEM((tpc, d), jnp.float32),
            pltpu.SemaphoreType.DMA,
        ],
    )
    def sc_gather(table_hbm, rid_hbm, pe_hbm, mod_hbm, out_hbm,
                  idx_v, rows_v, mod_v, pe_v, sem):
        wid = lax.axis_index("s") * NC + lax.axis_index("c")
        pltpu.sync_copy(mod_hbm, mod_v)
        row0 = wid * rpw
        pe0 = (wid * tpw) % s

        @pl.loop(0, nch)
        def _chunk(ci):
            base = row0 + ci * ch
            pltpu.sync_copy(rid_hbm.at[pl.ds(base, ch)], idx_v)
            pltpu.async_copy(table_hbm.at[idx_v], rows_v, sem).wait()
            pltpu.sync_copy(pe_hbm.at[pl.ds(pe0 + ci * tpc, tpc)], pe_v)

            @pl.loop(0, ch)
            def _row(rloc):
                t = rloc // (r + 1)
                j = rloc - t * (r + 1)

                @pl.loop(0, d // L)
                def _col(cc):
                    sl = pl.ds(cc * L, L)
                    rows_v.at[rloc, sl][...] = (
                        rows_v.at[rloc, sl][...]
                        + pe_v.at[t, sl][...]
                        + mod_v.at[j, sl][...]
                    )

            pltpu.sync_copy(rows_v, out_hbm.at[pl.ds(base, ch)])

    out = sc_gather(table, rowids, pe, mod9)
    return out.reshape(b, s, r + 1, d)


# speedup vs baseline: 6.3624x; 6.3624x over previous
import numpy as np
import jax
import jax.numpy as jnp
from jax.experimental import pallas as pl


def _pe_table(seq_len, d_model):
    position = np.arange(seq_len, dtype=np.float32)[:, None]
    div_term = np.exp(
        np.arange(0, d_model, 2, dtype=np.float32) * (-np.log(10000.0) / d_model)
    )
    pe = np.zeros((seq_len, d_model), dtype=np.float32)
    pe[:, 0::2] = np.sin(position * div_term)
    pe[:, 1::2] = np.cos(position * div_term)
    return pe


def _revert_body(tb_ref, idx_ref, pe_ref, mod_ref, mask_ref, out_ref):
    ts = pe_ref.shape[0]
    d = pe_ref.shape[1]
    pe_b = pe_ref[...]
    rows = [tb_ref[:, m, :] for m in range(5)]
    mask_b = jnp.broadcast_to(mask_ref[0:1, :], (ts, d))
    out_ref[:, 0, :] = rows[0] + pe_b + mod_ref[0:1, :]
    for j in range(1, 9):
        ij = idx_ref[:, j - 1 : j]
        v = mask_b
        for m in range(4):
            v = jnp.where(ij == m, rows[1 + m], v)
        out_ref[:, j, :] = v + pe_b + mod_ref[j : j + 1, :]


def kernel(temporal_block, mod_emb_weight, mask_token, temporal_revert_idx,
           temporal_masked_idx):
    b, s, m1, d = temporal_block.shape
    r = temporal_revert_idx.shape[-1]
    n = b * s

    tb = temporal_block.reshape(n, m1, d)
    idx = temporal_revert_idx.reshape(n, r).astype(jnp.int32)
    pe = jnp.asarray(_pe_table(s, d))
    mod9 = mod_emb_weight[: r + 1]

    ts = 128
    grid = (n // ts,)
    out = pl.pallas_call(
        _revert_body,
        grid=grid,
        in_specs=[
            pl.BlockSpec((ts, m1, d), lambda i: (i, 0, 0)),
            pl.BlockSpec((ts, r), lambda i: (i, 0)),
            pl.BlockSpec((ts, d), lambda i: (i % (s // ts), 0)),
            pl.BlockSpec((r + 1, d), lambda i: (0, 0)),
            pl.BlockSpec((1, d), lambda i: (0, 0)),
        ],
        out_specs=pl.BlockSpec((ts, r + 1, d), lambda i: (i, 0, 0)),
        out_shape=jax.ShapeDtypeStruct((n, r + 1, d), jnp.float32),
    )(tb, idx, pe, mod9, mask_token)
    return out.reshape(b, s, r + 1, d)


# speedup vs baseline: 6.5787x; 1.0340x over previous
import numpy as np
import jax
import jax.numpy as jnp
from jax.experimental import pallas as pl


def _pe_table(seq_len, d_model):
    position = np.arange(seq_len, dtype=np.float32)[:, None]
    div_term = np.exp(
        np.arange(0, d_model, 2, dtype=np.float32) * (-np.log(10000.0) / d_model)
    )
    pe = np.zeros((seq_len, d_model), dtype=np.float32)
    pe[:, 0::2] = np.sin(position * div_term)
    pe[:, 1::2] = np.cos(position * div_term)
    return pe


def _revert_body(tb_ref, idx_ref, pe_ref, mod_ref, mask_ref, out_ref):
    ts = pe_ref.shape[0]
    d = pe_ref.shape[1]
    pe_b = pe_ref[...]
    rows = [tb_ref[:, m, :] for m in range(5)]
    mask_b = jnp.broadcast_to(mask_ref[0:1, :], (ts, d))
    out_ref[:, 0, :] = rows[0] + pe_b + mod_ref[0:1, :]
    for j in range(1, 9):
        ij = idx_ref[:, j - 1 : j]
        v = mask_b
        for m in range(4):
            v = jnp.where(ij == m, rows[1 + m], v)
        out_ref[:, j, :] = v + pe_b + mod_ref[j : j + 1, :]


def kernel(temporal_block, mod_emb_weight, mask_token, temporal_revert_idx,
           temporal_masked_idx):
    b, s, m1, d = temporal_block.shape
    r = temporal_revert_idx.shape[-1]
    n = b * s

    tb = temporal_block.reshape(n, m1, d)
    idx = temporal_revert_idx.reshape(n, r).astype(jnp.int32)
    pe = jnp.asarray(_pe_table(s, d))
    mod9 = mod_emb_weight[: r + 1]

    ts = 256
    grid = (n // ts,)
    out = pl.pallas_call(
        _revert_body,
        grid=grid,
        in_specs=[
            pl.BlockSpec((ts, m1, d), lambda i: (i, 0, 0)),
            pl.BlockSpec((ts, r), lambda i: (i, 0)),
            pl.BlockSpec((ts, d), lambda i: (i % (s // ts), 0)),
            pl.BlockSpec((r + 1, d), lambda i: (0, 0)),
            pl.BlockSpec((1, d), lambda i: (0, 0)),
        ],
        out_specs=pl.BlockSpec((ts, r + 1, d), lambda i: (i, 0, 0)),
        out_shape=jax.ShapeDtypeStruct((n, r + 1, d), jnp.float32),
    )(tb, idx, pe, mod9, mask_token)
    return out.reshape(b, s, r + 1, d)
